# baseline (speedup 1.0000x reference)
import jax
import jax.numpy as jnp
import numpy as np
from jax.experimental import pallas as pl
from jax.experimental.pallas import tpu as pltpu

_B, _V = 128, 100000
_RB = 8
_CW = 2048
_NR = _B // _RB
_NC = (_V + _CW - 1) // _CW

_K0 = np.uint32(0)
_K1 = np.uint32(42)
_K2 = np.uint32(_K0 ^ _K1 ^ np.uint32(0x1BD11BDA))
_KS = (_K0, _K1, _K2)
_ROT = ((13, 15, 26, 6), (17, 29, 16, 24))
_TINY = np.float32(np.finfo(np.float32).tiny)
_IMAX = np.int32(2**31 - 1)


def _threefry_bits(flat_u32):
    x1 = flat_u32 + _K1
    x0 = jnp.zeros_like(x1) + _K0
    for it in range(5):
        for r in _ROT[it % 2]:
            x0 = x0 + x1
            x1 = (x1 << r) | (x1 >> (32 - r))
            x1 = x0 ^ x1
        x0 = x0 + _KS[(it + 1) % 3]
        x1 = x1 + _KS[(it + 2) % 3] + np.uint32(it + 1)
    return x0 ^ x1


def _sample_kernel(logits_ref, out_ref, vmax_ref, vidx_ref):
    i = pl.program_id(0)
    j = pl.program_id(1)

    @pl.when(j == 0)
    def _init():
        vmax_ref[...] = jnp.full((_RB, _CW), -jnp.inf, jnp.float32)
        vidx_ref[...] = jnp.zeros((_RB, _CW), jnp.int32)

    rowi = jax.lax.broadcasted_iota(jnp.int32, (_RB, _CW), 0) + i * _RB
    coli = jax.lax.broadcasted_iota(jnp.int32, (_RB, _CW), 1) + j * _CW
    flat = rowi * _V + coli

    bits = _threefry_bits(flat.astype(jnp.uint32))
    fb = (bits >> 9) | jnp.uint32(0x3F800000)
    u = jax.lax.bitcast_convert_type(fb, jnp.float32) - jnp.float32(1.0)
    u = jnp.maximum(u, _TINY)
    gumbel = -jnp.log(-jnp.log(u))

    val = gumbel + logits_ref[...]
    val = jnp.where(coli < _V, val, -jnp.inf)

    cond = val > vmax_ref[...]
    vmax_ref[...] = jnp.where(cond, val, vmax_ref[...])
    vidx_ref[...] = jnp.where(cond, coli, vidx_ref[...])

    @pl.when(j == _NC - 1)
    def _finish():
        vm = vmax_ref[...]
        rmax = jnp.max(vm, axis=1, keepdims=True)
        cand = jnp.where(vm == rmax, vidx_ref[...], _IMAX)
        out_ref[0, 0, :] = jnp.min(cand, axis=1)


def kernel(logits):
    out = pl.pallas_call(
        _sample_kernel,
        grid=(_NR, _NC),
        in_specs=[pl.BlockSpec((_RB, _CW), lambda i, j: (i, j))],
        out_specs=pl.BlockSpec((1, 1, _RB), lambda i, j: (i, 0, 0)),
        out_shape=jax.ShapeDtypeStruct((_NR, 1, _RB), jnp.int32),
        scratch_shapes=[
            pltpu.VMEM((_RB, _CW), jnp.float32),
            pltpu.VMEM((_RB, _CW), jnp.int32),
        ],
        compiler_params=pltpu.CompilerParams(
            dimension_semantics=("parallel", "arbitrary"),
        ),
    )(logits)
    return out.reshape(_B)


# speedup vs baseline: 2.0101x; 2.0101x over previous
import jax
import jax.numpy as jnp
import numpy as np
from jax.experimental import pallas as pl
from jax.experimental.pallas import tpu as pltpu

_B, _V = 128, 100000
_RB = 8
_SW = 1024
_NS = (_V + _SW - 1) // _SW
_CW = _NS * _SW
_NR = _B // _RB

_K1 = np.uint32(42)
_K2 = np.uint32(0 ^ 42 ^ 0x1BD11BDA)
_KS = (np.uint32(0), _K1, _K2)
_ROT = ((13, 15, 26, 6), (17, 29, 16, 24))
_TINY = np.float32(np.finfo(np.float32).tiny)
_IMAX = np.int32(2**31 - 1)


def _threefry_bits(x1):
    x0 = x1
    x1 = x1 ^ ((x1 << 13) | (x1 >> 19))
    for r in _ROT[0][1:]:
        x0 = x0 + x1
        x1 = (x1 << r) | (x1 >> (32 - r))
        x1 = x0 ^ x1
    x0 = x0 + _KS[1]
    x1 = x1 + _KS[2] + np.uint32(1)
    for it in range(1, 5):
        for r in _ROT[it % 2]:
            x0 = x0 + x1
            x1 = (x1 << r) | (x1 >> (32 - r))
            x1 = x0 ^ x1
        x0 = x0 + _KS[(it + 1) % 3]
        x1 = x1 + _KS[(it + 2) % 3] + np.uint32(it + 1)
    return x0 ^ x1


def _gumbel_plus(logits, x1):
    bits = _threefry_bits(x1)
    fb = (bits >> 9) | jnp.uint32(0x3F800000)
    u = jax.lax.bitcast_convert_type(fb, jnp.float32) - jnp.float32(1.0)
    u = jnp.maximum(u, _TINY)
    return logits - jnp.log(-jnp.log(u))


def _sample_kernel(logits_ref, out_ref):
    i = pl.program_id(0)

    pat0 = (jax.lax.broadcasted_iota(jnp.int32, (_RB, _SW), 0) * _V
            + jax.lax.broadcasted_iota(jnp.int32, (_RB, _SW), 1))
    pat0 = pat0.astype(jnp.uint32)
    base0 = (i * (_RB * _V) + 42).astype(jnp.uint32)

    vmax = jnp.full((_RB, _SW), -jnp.inf, jnp.float32)
    vidx = jnp.zeros((_RB, _SW), jnp.int32)
    for k in range(_NS):
        x1 = pat0 + (base0 + np.uint32(k * _SW))
        val = _gumbel_plus(logits_ref[:, pl.ds(k * _SW, _SW)], x1)
        if k == _NS - 1:
            colp = jax.lax.broadcasted_iota(jnp.int32, (_RB, _SW), 1)
            val = jnp.where(colp < _V - (_NS - 1) * _SW, val, -jnp.inf)
        vidx = jnp.where(val > vmax, k, vidx)
        vmax = jnp.maximum(vmax, val)

    colp = jax.lax.broadcasted_iota(jnp.int32, (_RB, _SW), 1)
    col = vidx * _SW + colp
    rmax = jnp.max(vmax, axis=1, keepdims=True)
    cand = jnp.where(vmax == rmax, col, _IMAX)
    out_ref[0, 0, :] = jnp.min(cand, axis=1)


def kernel(logits):
    out = pl.pallas_call(
        _sample_kernel,
        grid=(_NR,),
        in_specs=[pl.BlockSpec((_RB, _CW), lambda i: (i, 0))],
        out_specs=pl.BlockSpec((1, 1, _RB), lambda i: (i, 0, 0)),
        out_shape=jax.ShapeDtypeStruct((_NR, 1, _RB), jnp.int32),
        compiler_params=pltpu.CompilerParams(
            dimension_semantics=("arbitrary",),
        ),
    )(logits)
    return out.reshape(_B)
